# Initial kernel scaffold; baseline (speedup 1.0000x reference)
import jax
import jax.numpy as jnp
from jax import lax
from jax.experimental import pallas as pl
from jax.experimental.pallas import tpu as pltpu
from jax.experimental.pallas import tpu_sc as plsc

N = 10000
D = 128
H = 64
E = 320000
NT = 16
NC = 2
CH = 128
NCHUNK = 158
EPT = NCHUNK * CH
AGG_R = 10240
PAD_ROW = N
RPT = 625
RC = 125
C1 = 0.5
C2 = 0.5

_f32 = jnp.float32


def _sc_body(xc_hbm, col_hbm, row_hbm, v_hbm,
             agg, degd, cidx, ridx, gbuf, ones16, zb16, zeros,
             degs, scalev, aggc, xcc, vc, sem):
    c = lax.axis_index("c")
    s = lax.axis_index("s")
    base_n = c * N + s * RPT
    abase = s * RPT
    zbase = s * 640

    pltpu.sync_copy(col_hbm.at[c, s], cidx)
    pltpu.sync_copy(row_hbm.at[s], ridx)

    @pl.loop(0, CH)
    def _fill(r):
        ones16[r, :] = jnp.ones((16,), _f32)
        zb16[r, :] = jnp.zeros((16,), _f32)
        for q in range(4):
            zeros[r, pl.ds(q * 16, 16)] = jnp.zeros((16,), _f32)

    @pl.loop(0, 5)
    def _z(k):
        pltpu.sync_copy(zeros, agg.at[pl.ds(zbase + k * CH, CH)])
        pltpu.sync_copy(zb16, degd.at[pl.ds(zbase + k * CH, CH)])

    plsc.subcore_barrier()

    @pl.loop(0, NCHUNK)
    def _deg(j):
        pltpu.sync_copy(ones16, degd.at[ridx.at[j]], add=True)

    plsc.subcore_barrier()

    @pl.loop(0, 5)
    def _scale(k):
        pltpu.sync_copy(degd.at[pl.ds(abase + k * RC, RC)], degs)

        @pl.loop(0, RC)
        def _srow(r):
            d = degs[r, :]
            scalev[k * RC + r, :] = 1.0 / (d + 1.0)

    @pl.loop(0, 5)
    def _init(k):
        pltpu.sync_copy(xc_hbm.at[pl.ds(base_n + k * RC, RC)], vc)
        pltpu.sync_copy(vc, v_hbm.at[pl.ds(base_n + k * RC, RC)])

    plsc.subcore_barrier()

    @pl.loop(0, 50)
    def _iter(it):
        @pl.loop(0, NCHUNK)
        def _edge(j):
            pltpu.async_copy(v_hbm.at[cidx.at[j]], gbuf, sem).wait()
            pltpu.sync_copy(gbuf, agg.at[ridx.at[j]], add=True)

        plsc.subcore_barrier()

        @pl.loop(0, 5)
        def _comb(k):
            pltpu.sync_copy(agg.at[pl.ds(abase + k * RC, RC)], aggc)
            pltpu.sync_copy(zeros.at[pl.ds(0, RC)],
                            agg.at[pl.ds(abase + k * RC, RC)])
            pltpu.sync_copy(xc_hbm.at[pl.ds(base_n + k * RC, RC)], xcc)
            pltpu.sync_copy(v_hbm.at[pl.ds(base_n + k * RC, RC)], vc)

            @pl.loop(0, RC)
            def _row(r):
                sc = scalev[k * RC + r, :]
                for q in range(4):
                    sl = pl.ds(q * 16, 16)
                    a = aggc[r, sl]
                    v = vc[r, sl]
                    x = xcc[r, sl]
                    vc[r, sl] = C1 * sc * (a + v) + C2 * x

            pltpu.sync_copy(vc, v_hbm.at[pl.ds(base_n + k * RC, RC)])

        plsc.subcore_barrier()


_sc_propagate = pl.kernel(
    _sc_body,
    out_type=jax.ShapeDtypeStruct((NC * N, H), _f32),
    mesh=plsc.VectorSubcoreMesh(core_axis_name="c", subcore_axis_name="s",
                                num_cores=NC, num_subcores=NT),
    scratch_types=[
        pltpu.VMEM_SHARED((AGG_R, H), _f32),
        pltpu.VMEM_SHARED((AGG_R, 16), _f32),
        pltpu.VMEM((NCHUNK, CH), jnp.int32),
        pltpu.VMEM((NCHUNK, CH), jnp.int32),
        pltpu.VMEM((CH, H), _f32),
        pltpu.VMEM((CH, 16), _f32),
        pltpu.VMEM((CH, 16), _f32),
        pltpu.VMEM((CH, H), _f32),
        pltpu.VMEM((RC, 16), _f32),
        pltpu.VMEM((RPT, 16), _f32),
        pltpu.VMEM((RC, H), _f32),
        pltpu.VMEM((RC, H), _f32),
        pltpu.VMEM((RC, H), _f32),
        pltpu.SemaphoreType.DMA,
    ],
)


def _colsum_body(x_ref, o_ref):
    @pl.when(pl.program_id(0) == 0)
    def _():
        o_ref[...] = jnp.zeros_like(o_ref)
    o_ref[...] += jnp.sum(x_ref[...], axis=0, keepdims=True)


def _center_body(x_ref, sum_ref, a_ref, b_ref):
    xc = x_ref[...] - sum_ref[...] * (1.0 / N)
    a_ref[...] = xc[:, :H]
    b_ref[...] = xc[:, H:]


def _matmul_body(v_ref, w_ref, b_ref, o_ref):
    o_ref[...] = (
        jnp.dot(v_ref[...], w_ref[...], preferred_element_type=_f32)
        + b_ref[...]
    )


def kernel(x, edge_index, y, train_mask, weight, bias):
    row = edge_index[0]
    col = edge_index[1]

    pad = NT * EPT - E
    col_t = jnp.pad(col.reshape(NT, E // NT), ((0, 0), (0, pad // NT))
                    ).reshape(NT, NCHUNK, CH)
    row_t = jnp.pad(row.reshape(NT, E // NT), ((0, 0), (0, pad // NT)),
                    constant_values=PAD_ROW).reshape(NT, NCHUNK, CH)
    col_both = jnp.stack([col_t, col_t + N])

    colsum = pl.pallas_call(
        _colsum_body,
        grid=(10,),
        in_specs=[pl.BlockSpec((N // 10, D), lambda i: (i, 0))],
        out_specs=pl.BlockSpec((1, D), lambda i: (0, 0)),
        out_shape=jax.ShapeDtypeStruct((1, D), _f32),
    )(x)
    xca, xcb = pl.pallas_call(
        _center_body,
        grid=(10,),
        in_specs=[pl.BlockSpec((N // 10, D), lambda i: (i, 0)),
                  pl.BlockSpec((1, D), lambda i: (0, 0))],
        out_specs=[pl.BlockSpec((N // 10, H), lambda i: (i, 0)),
                   pl.BlockSpec((N // 10, H), lambda i: (i, 0))],
        out_shape=[jax.ShapeDtypeStruct((N, H), _f32),
                   jax.ShapeDtypeStruct((N, H), _f32)],
    )(x, colsum)
    xc_flat = jnp.concatenate([xca, xcb], axis=0)

    v_flat = _sc_propagate(xc_flat, col_both, row_t)
    v = jnp.concatenate([v_flat[:N], v_flat[N:]], axis=1)

    out = pl.pallas_call(
        _matmul_body,
        grid=(10,),
        in_specs=[pl.BlockSpec((N // 10, D), lambda i: (i, 0)),
                  pl.BlockSpec((D, D), lambda i: (0, 0)),
                  pl.BlockSpec((1, D), lambda i: (0, 0))],
        out_specs=pl.BlockSpec((N // 10, D), lambda i: (i, 0)),
        out_shape=jax.ShapeDtypeStruct((N, D), _f32),
    )(v, weight, bias)
    return out


# speedup vs baseline: 8.1227x; 8.1227x over previous
import jax
import jax.numpy as jnp
from jax import lax
from jax.experimental import pallas as pl
from jax.experimental.pallas import tpu as pltpu
from jax.experimental.pallas import tpu_sc as plsc

N = 10000
NP = 10240
D = 128
H = 64
E = 320000
NT = 16
NC = 2
CH = 128
NCHUNK = 158
EPT = NCHUNK * CH
AGG_R = NP
PAD_ROW = N
RPT = 640
RC = 64
NKC = RPT // RC
C1 = 0.5
C2 = 0.5

_f32 = jnp.float32


def _sc_body(xc_hbm, col_hbm, row_hbm, v_hbm,
             agg, cidx, ridx, gbuf, zeros, scalev, aggc, xcc, vc, sem):
    c = lax.axis_index("c")
    s = lax.axis_index("s")
    base_n = c * NP + s * RPT
    abase = s * RPT

    pltpu.sync_copy(col_hbm.at[c, s], cidx)
    pltpu.sync_copy(row_hbm.at[s], ridx)

    @pl.loop(0, RC)
    def _fillz(r):
        for q in range(4):
            zeros[r, pl.ds(q * 16, 16)] = jnp.zeros((16,), _f32)

    @pl.loop(0, CH)
    def _fillo(r):
        for q in range(4):
            gbuf[r, pl.ds(q * 16, 16)] = jnp.ones((16,), _f32)

    @pl.loop(0, NKC)
    def _z(k):
        pltpu.sync_copy(zeros, agg.at[pl.ds(abase + k * RC, RC)])

    plsc.subcore_barrier()

    @pl.loop(0, NCHUNK)
    def _deg(j):
        pltpu.sync_copy(gbuf, agg.at[ridx.at[j]], add=True)

    plsc.subcore_barrier()

    @pl.loop(0, NKC)
    def _scale(k):
        pltpu.sync_copy(agg.at[pl.ds(abase + k * RC, RC)], aggc)
        pltpu.sync_copy(zeros, agg.at[pl.ds(abase + k * RC, RC)])

        @pl.loop(0, RC)
        def _srow(r):
            d = aggc[r, pl.ds(0, 16)]
            scalev[k * RC + r, :] = 1.0 / (d + 1.0)

        pltpu.sync_copy(xc_hbm.at[pl.ds(base_n + k * RC, RC)], vc)
        pltpu.sync_copy(vc, v_hbm.at[pl.ds(base_n + k * RC, RC)])

    plsc.subcore_barrier()

    @pl.loop(0, 50)
    def _iter(it):
        @pl.loop(0, NCHUNK)
        def _edge(j):
            pltpu.async_copy(v_hbm.at[cidx.at[j]], gbuf, sem).wait()
            pltpu.sync_copy(gbuf, agg.at[ridx.at[j]], add=True)

        plsc.subcore_barrier()

        @pl.loop(0, NKC)
        def _comb(k):
            pltpu.sync_copy(agg.at[pl.ds(abase + k * RC, RC)], aggc)
            pltpu.sync_copy(zeros, agg.at[pl.ds(abase + k * RC, RC)])
            pltpu.sync_copy(xc_hbm.at[pl.ds(base_n + k * RC, RC)], xcc)
            pltpu.sync_copy(v_hbm.at[pl.ds(base_n + k * RC, RC)], vc)

            @pl.loop(0, RC)
            def _row(r):
                sc = scalev[k * RC + r, :]
                for q in range(4):
                    sl = pl.ds(q * 16, 16)
                    a = aggc[r, sl]
                    v = vc[r, sl]
                    x = xcc[r, sl]
                    vc[r, sl] = C1 * sc * (a + v) + C2 * x

            pltpu.sync_copy(vc, v_hbm.at[pl.ds(base_n + k * RC, RC)])

        plsc.subcore_barrier()


_sc_propagate = pl.kernel(
    _sc_body,
    out_type=jax.ShapeDtypeStruct((NC * NP, H), _f32),
    mesh=plsc.VectorSubcoreMesh(core_axis_name="c", subcore_axis_name="s",
                                num_cores=NC, num_subcores=NT),
    compiler_params=pltpu.CompilerParams(use_tc_tiling_on_sc=False),
    scratch_types=[
        pltpu.VMEM_SHARED((AGG_R, H), _f32),
        pltpu.VMEM((NCHUNK, CH), jnp.int32),
        pltpu.VMEM((NCHUNK, CH), jnp.int32),
        pltpu.VMEM((CH, H), _f32),
        pltpu.VMEM((RC, H), _f32),
        pltpu.VMEM((RPT, 16), _f32),
        pltpu.VMEM((RC, H), _f32),
        pltpu.VMEM((RC, H), _f32),
        pltpu.VMEM((RC, H), _f32),
        pltpu.SemaphoreType.DMA,
    ],
)


def _colsum_body(x_ref, o_ref):
    @pl.when(pl.program_id(0) == 0)
    def _():
        o_ref[...] = jnp.zeros_like(o_ref)
    o_ref[...] += jnp.sum(x_ref[...], axis=0, keepdims=True)


def _center_body(x_ref, sum_ref, a_ref, b_ref):
    xc = x_ref[...] - sum_ref[...] * (1.0 / N)
    a_ref[...] = xc[:, :H]
    b_ref[...] = xc[:, H:]


def _matmul_body(v_ref, w_ref, b_ref, o_ref):
    o_ref[...] = (
        jnp.dot(v_ref[...], w_ref[...], preferred_element_type=_f32)
        + b_ref[...]
    )


def kernel(x, edge_index, y, train_mask, weight, bias):
    row = edge_index[0]
    col = edge_index[1]

    pad = NT * EPT - E
    col_t = jnp.pad(col.reshape(NT, E // NT), ((0, 0), (0, pad // NT))
                    ).reshape(NT, NCHUNK, CH)
    row_t = jnp.pad(row.reshape(NT, E // NT), ((0, 0), (0, pad // NT)),
                    constant_values=PAD_ROW).reshape(NT, NCHUNK, CH)
    col_both = jnp.stack([col_t, col_t + NP])

    colsum = pl.pallas_call(
        _colsum_body,
        grid=(10,),
        in_specs=[pl.BlockSpec((N // 10, D), lambda i: (i, 0))],
        out_specs=pl.BlockSpec((1, D), lambda i: (0, 0)),
        out_shape=jax.ShapeDtypeStruct((1, D), _f32),
    )(x)
    xca, xcb = pl.pallas_call(
        _center_body,
        grid=(10,),
        in_specs=[pl.BlockSpec((N // 10, D), lambda i: (i, 0)),
                  pl.BlockSpec((1, D), lambda i: (0, 0))],
        out_specs=[pl.BlockSpec((N // 10, H), lambda i: (i, 0)),
                   pl.BlockSpec((N // 10, H), lambda i: (i, 0))],
        out_shape=[jax.ShapeDtypeStruct((N, H), _f32),
                   jax.ShapeDtypeStruct((N, H), _f32)],
    )(x, colsum)
    zpad = jnp.zeros((NP - N, H), _f32)
    xc_flat = jnp.concatenate([xca, zpad, xcb, zpad], axis=0)

    v_flat = _sc_propagate(xc_flat, col_both, row_t)
    v = jnp.concatenate([v_flat[:N], v_flat[NP:NP + N]], axis=1)

    out = pl.pallas_call(
        _matmul_body,
        grid=(10,),
        in_specs=[pl.BlockSpec((N // 10, D), lambda i: (i, 0)),
                  pl.BlockSpec((D, D), lambda i: (0, 0)),
                  pl.BlockSpec((1, D), lambda i: (0, 0))],
        out_specs=pl.BlockSpec((N // 10, D), lambda i: (i, 0)),
        out_shape=jax.ShapeDtypeStruct((N, D), _f32),
    )(v, weight, bias)
    return out


# speedup vs baseline: 12.5189x; 1.5412x over previous
import jax
import jax.numpy as jnp
from jax import lax
from jax.experimental import pallas as pl
from jax.experimental.pallas import tpu as pltpu
from jax.experimental.pallas import tpu_sc as plsc

N = 10000
NP = 10240
D = 128
H = 64
E = 320000
NT = 16
NC = 2
CH = 128
NCHUNK = 158
EPT = NCHUNK * CH
AGG_R = NP
PAD_ROW = N
RPT = 640
RC = 64
NKC = RPT // RC
C1 = 0.5
C2 = 0.5

_f32 = jnp.float32


def _sc_body(xc_hbm, col_hbm, row_hbm, v_hbm,
             agg, cidx, ridx, gbuf0, gbuf1, zeros, scalev, aggc, xcc, vc,
             gsem0, gsem1):
    c = lax.axis_index("c")
    s = lax.axis_index("s")
    base_n = c * NP + s * RPT
    abase = s * RPT

    pltpu.sync_copy(col_hbm.at[c, s], cidx)
    pltpu.sync_copy(row_hbm.at[s], ridx)

    @pl.loop(0, RC)
    def _fillz(r):
        for q in range(4):
            zeros[r, pl.ds(q * 16, 16)] = jnp.zeros((16,), _f32)

    @pl.loop(0, CH)
    def _fillo(r):
        for q in range(4):
            gbuf0[r, pl.ds(q * 16, 16)] = jnp.ones((16,), _f32)

    @pl.loop(0, NKC)
    def _z(k):
        pltpu.sync_copy(zeros, agg.at[pl.ds(abase + k * RC, RC)])

    plsc.subcore_barrier()

    @pl.loop(0, NCHUNK)
    def _deg(j):
        pltpu.sync_copy(gbuf0, agg.at[ridx.at[j]], add=True)

    plsc.subcore_barrier()

    @pl.loop(0, NKC)
    def _scale(k):
        pltpu.sync_copy(agg.at[pl.ds(abase + k * RC, RC)], aggc)
        pltpu.sync_copy(zeros, agg.at[pl.ds(abase + k * RC, RC)])

        @pl.loop(0, RC)
        def _srow(r):
            d = aggc[r, pl.ds(0, 16)]
            scalev[k * RC + r, :] = 1.0 / (d + 1.0)

        pltpu.sync_copy(xc_hbm.at[pl.ds(base_n + k * RC, RC)], vc)
        pltpu.sync_copy(vc, v_hbm.at[pl.ds(base_n + k * RC, RC)])

    plsc.subcore_barrier()

    @pl.loop(0, 50)
    def _iter(it):
        pltpu.async_copy(v_hbm.at[cidx.at[0]], gbuf0, gsem0)
        pltpu.async_copy(v_hbm.at[cidx.at[1]], gbuf1, gsem1)

        @pl.loop(0, NCHUNK, step=2)
        def _edge(j):
            pltpu.make_async_copy(v_hbm.at[pl.ds(0, CH)], gbuf0, gsem0).wait()
            pltpu.sync_copy(gbuf0, agg.at[ridx.at[j]], add=True)

            @pl.when(j + 2 < NCHUNK)
            def _():
                pltpu.async_copy(v_hbm.at[cidx.at[j + 2]], gbuf0, gsem0)

            pltpu.make_async_copy(v_hbm.at[pl.ds(0, CH)], gbuf1, gsem1).wait()
            pltpu.sync_copy(gbuf1, agg.at[ridx.at[j + 1]], add=True)

            @pl.when(j + 3 < NCHUNK)
            def _():
                pltpu.async_copy(v_hbm.at[cidx.at[j + 3]], gbuf1, gsem1)

        plsc.subcore_barrier()

        @pl.loop(0, NKC)
        def _comb(k):
            pltpu.sync_copy(agg.at[pl.ds(abase + k * RC, RC)], aggc)
            pltpu.sync_copy(zeros, agg.at[pl.ds(abase + k * RC, RC)])
            pltpu.sync_copy(xc_hbm.at[pl.ds(base_n + k * RC, RC)], xcc)
            pltpu.sync_copy(v_hbm.at[pl.ds(base_n + k * RC, RC)], vc)

            @pl.loop(0, RC)
            def _row(r):
                sc = scalev[k * RC + r, :]
                for q in range(4):
                    sl = pl.ds(q * 16, 16)
                    a = aggc[r, sl]
                    v = vc[r, sl]
                    x = xcc[r, sl]
                    vc[r, sl] = C1 * sc * (a + v) + C2 * x

            pltpu.sync_copy(vc, v_hbm.at[pl.ds(base_n + k * RC, RC)])

        plsc.subcore_barrier()


_sc_propagate = pl.kernel(
    _sc_body,
    out_type=jax.ShapeDtypeStruct((NC * NP, H), _f32),
    mesh=plsc.VectorSubcoreMesh(core_axis_name="c", subcore_axis_name="s",
                                num_cores=NC, num_subcores=NT),
    compiler_params=pltpu.CompilerParams(use_tc_tiling_on_sc=False),
    scratch_types=[
        pltpu.VMEM_SHARED((AGG_R, H), _f32),
        pltpu.VMEM((NCHUNK, CH), jnp.int32),
        pltpu.VMEM((NCHUNK, CH), jnp.int32),
        pltpu.VMEM((CH, H), _f32),
        pltpu.VMEM((CH, H), _f32),
        pltpu.VMEM((RC, H), _f32),
        pltpu.VMEM((RPT, 16), _f32),
        pltpu.VMEM((RC, H), _f32),
        pltpu.VMEM((RC, H), _f32),
        pltpu.VMEM((RC, H), _f32),
        pltpu.SemaphoreType.DMA,
        pltpu.SemaphoreType.DMA,
    ],
)


def _colsum_body(x_ref, o_ref):
    @pl.when(pl.program_id(0) == 0)
    def _():
        o_ref[...] = jnp.zeros_like(o_ref)
    o_ref[...] += jnp.sum(x_ref[...], axis=0, keepdims=True)


def _center_body(x_ref, sum_ref, a_ref, b_ref):
    xc = x_ref[...] - sum_ref[...] * (1.0 / N)
    a_ref[...] = xc[:, :H]
    b_ref[...] = xc[:, H:]


def _matmul_body(v_ref, w_ref, b_ref, o_ref):
    o_ref[...] = (
        jnp.dot(v_ref[...], w_ref[...], preferred_element_type=_f32)
        + b_ref[...]
    )


def kernel(x, edge_index, y, train_mask, weight, bias):
    row = edge_index[0]
    col = edge_index[1]

    pad = NT * EPT - E
    col_t = jnp.pad(col.reshape(NT, E // NT), ((0, 0), (0, pad // NT))
                    ).reshape(NT, NCHUNK, CH)
    row_t = jnp.pad(row.reshape(NT, E // NT), ((0, 0), (0, pad // NT)),
                    constant_values=PAD_ROW).reshape(NT, NCHUNK, CH)
    col_both = jnp.stack([col_t, col_t + NP])

    colsum = pl.pallas_call(
        _colsum_body,
        grid=(10,),
        in_specs=[pl.BlockSpec((N // 10, D), lambda i: (i, 0))],
        out_specs=pl.BlockSpec((1, D), lambda i: (0, 0)),
        out_shape=jax.ShapeDtypeStruct((1, D), _f32),
    )(x)
    xca, xcb = pl.pallas_call(
        _center_body,
        grid=(10,),
        in_specs=[pl.BlockSpec((N // 10, D), lambda i: (i, 0)),
                  pl.BlockSpec((1, D), lambda i: (0, 0))],
        out_specs=[pl.BlockSpec((N // 10, H), lambda i: (i, 0)),
                   pl.BlockSpec((N // 10, H), lambda i: (i, 0))],
        out_shape=[jax.ShapeDtypeStruct((N, H), _f32),
                   jax.ShapeDtypeStruct((N, H), _f32)],
    )(x, colsum)
    zpad = jnp.zeros((NP - N, H), _f32)
    xc_flat = jnp.concatenate([xca, zpad, xcb, zpad], axis=0)

    v_flat = _sc_propagate(xc_flat, col_both, row_t)
    v = jnp.concatenate([v_flat[:N], v_flat[NP:NP + N]], axis=1)

    out = pl.pallas_call(
        _matmul_body,
        grid=(10,),
        in_specs=[pl.BlockSpec((N // 10, D), lambda i: (i, 0)),
                  pl.BlockSpec((D, D), lambda i: (0, 0)),
                  pl.BlockSpec((1, D), lambda i: (0, 0))],
        out_specs=pl.BlockSpec((N // 10, D), lambda i: (i, 0)),
        out_shape=jax.ShapeDtypeStruct((N, D), _f32),
    )(v, weight, bias)
    return out
